# baseline (speedup 1.0000x reference)
import jax
import jax.numpy as jnp
from jax.experimental import pallas as pl

_BLOCK = 2048


def _dot_rt(lhs, rhs):
    return jax.lax.dot_general(lhs, rhs, (((1,), (1,)), ((), ())),
                               preferred_element_type=jnp.float32)


def _attr_block(vbt_ref, tw_ref, de_ref, a12_ref, w3_ref, w4_ref,
                m3_ref, m4_ref, c_ref, o_ref):
    h3 = _dot_rt(tw_ref[...], w3_ref[...])
    acc = _dot_rt(h3, m3_ref[...])
    h4 = _dot_rt(de_ref[...], w4_ref[...])
    acc = acc + _dot_rt(h4, m4_ref[...])
    acc = acc + jax.lax.dot_general(vbt_ref[...], a12_ref[...],
                                    (((0,), (0,)), ((), ())),
                                    preferred_element_type=jnp.float32)
    acc = acc + c_ref[...]
    o_ref[...] = jnp.where(acc >= 0.0, acc, 0.01 * acc)


def kernel(value_feats, bool_feats, tweet_feats, des_feats,
           W1, b1, W2, b2, W3, b3, W4, b4,
           W_in, b_in, W_out, b_out, W_r, b_r):
    N, VN = value_feats.shape
    BN = bool_feats.shape[1]
    TN = tweet_feats.shape[1]
    DN = des_feats.shape[1]
    FD = W_r.shape[0]
    E = W_out.shape[0]

    Wv = W_in[2 * E:3 * E]
    bv = b_in[2 * E:3 * E]
    m_t = W_r @ W_out @ Wv
    a1 = (m_t[:, 0 * FD:1 * FD] @ W1).T
    a2 = (m_t[:, 1 * FD:2 * FD] @ W2).T
    a12 = jnp.concatenate([a1, a2], axis=0)
    m3 = m_t[:, 2 * FD:3 * FD]
    m4 = m_t[:, 3 * FD:4 * FD]
    bx = jnp.concatenate([b1, b2, b3, b4])
    c = bx @ m_t.T + bv @ (W_r @ W_out).T + b_out @ W_r.T + b_r
    c2 = c.reshape(1, FD)
    vbt = jnp.concatenate([value_feats, bool_feats], axis=1).T

    grid = (pl.cdiv(N, _BLOCK),)
    out = pl.pallas_call(
        _attr_block,
        grid=grid,
        in_specs=[
            pl.BlockSpec((VN + BN, _BLOCK), lambda i: (0, i)),
            pl.BlockSpec((_BLOCK, TN), lambda i: (i, 0)),
            pl.BlockSpec((_BLOCK, DN), lambda i: (i, 0)),
            pl.BlockSpec((VN + BN, FD), lambda i: (0, 0)),
            pl.BlockSpec((FD, TN), lambda i: (0, 0)),
            pl.BlockSpec((FD, DN), lambda i: (0, 0)),
            pl.BlockSpec((FD, FD), lambda i: (0, 0)),
            pl.BlockSpec((FD, FD), lambda i: (0, 0)),
            pl.BlockSpec((1, FD), lambda i: (0, 0)),
        ],
        out_specs=pl.BlockSpec((_BLOCK, FD), lambda i: (i, 0)),
        out_shape=jax.ShapeDtypeStruct((N, FD), jnp.float32),
    )(vbt, tweet_feats, des_feats, a12, W3, W4, m3, m4, c2)
    return out


# speedup vs baseline: 1.0332x; 1.0332x over previous
import jax
import jax.numpy as jnp
from jax.experimental import pallas as pl

_BLOCK = 2560


def _attr_block(vbt_ref, tw_ref, de_ref, a12_ref, a3_ref, a4_ref, c_ref, o_ref):
    acc = jnp.dot(tw_ref[...], a3_ref[...], preferred_element_type=jnp.float32)
    acc = acc + jnp.dot(de_ref[...], a4_ref[...], preferred_element_type=jnp.float32)
    acc = acc + jax.lax.dot_general(vbt_ref[...], a12_ref[...],
                                    (((0,), (0,)), ((), ())),
                                    preferred_element_type=jnp.float32)
    acc = acc + c_ref[...]
    o_ref[...] = jnp.where(acc >= 0.0, acc, 0.01 * acc)


def kernel(value_feats, bool_feats, tweet_feats, des_feats,
           W1, b1, W2, b2, W3, b3, W4, b4,
           W_in, b_in, W_out, b_out, W_r, b_r):
    N, VN = value_feats.shape
    BN = bool_feats.shape[1]
    TN = tweet_feats.shape[1]
    DN = des_feats.shape[1]
    FD = W_r.shape[0]
    E = W_out.shape[0]

    Wv = W_in[2 * E:3 * E]
    bv = b_in[2 * E:3 * E]
    m_t = W_r @ W_out @ Wv
    a1 = (m_t[:, 0 * FD:1 * FD] @ W1).T
    a2 = (m_t[:, 1 * FD:2 * FD] @ W2).T
    a3 = (m_t[:, 2 * FD:3 * FD] @ W3).T
    a4 = (m_t[:, 3 * FD:4 * FD] @ W4).T
    a12 = jnp.concatenate([a1, a2], axis=0)
    bx = jnp.concatenate([b1, b2, b3, b4])
    c = bx @ m_t.T + bv @ (W_r @ W_out).T + b_out @ W_r.T + b_r
    c2 = c.reshape(1, FD)
    vbt = jnp.concatenate([value_feats, bool_feats], axis=1).T

    grid = (pl.cdiv(N, _BLOCK),)
    out = pl.pallas_call(
        _attr_block,
        grid=grid,
        in_specs=[
            pl.BlockSpec((VN + BN, _BLOCK), lambda i: (0, i)),
            pl.BlockSpec((_BLOCK, TN), lambda i: (i, 0)),
            pl.BlockSpec((_BLOCK, DN), lambda i: (i, 0)),
            pl.BlockSpec((VN + BN, FD), lambda i: (0, 0)),
            pl.BlockSpec((TN, FD), lambda i: (0, 0)),
            pl.BlockSpec((DN, FD), lambda i: (0, 0)),
            pl.BlockSpec((1, FD), lambda i: (0, 0)),
        ],
        out_specs=pl.BlockSpec((_BLOCK, FD), lambda i: (i, 0)),
        out_shape=jax.ShapeDtypeStruct((N, FD), jnp.float32),
    )(vbt, tweet_feats, des_feats, a12, a3, a4, c2)
    return out
